# baseline (speedup 1.0000x reference)
import functools

import jax
import jax.numpy as jnp
from jax import lax
from jax.experimental import pallas as pl
from jax.experimental.pallas import tpu as pltpu
from jax.experimental.pallas import tpu_sc as plsc

LANES = 16


def kernel(x, tables):
    B, L = x.shape
    D = tables.shape[2]
    info = plsc.get_sparse_core_info()
    NC, NS = info.num_cores, info.num_subcores
    NW = NC * NS
    BW = B // NW
    CB = 16
    NCH = BW // CB
    LP = 112
    NVR = LP // LANES

    flat_tab = tables.reshape(L * D, D)
    x_pad = jnp.pad(x.astype(jnp.int32), ((0, 0), (0, LP - L)))
    off = jnp.where(jnp.arange(LP) < L,
                    jnp.arange(LP, dtype=jnp.int32) * D, 0).astype(jnp.int32)

    mesh = plsc.VectorSubcoreMesh(core_axis_name="c", subcore_axis_name="s")

    @functools.partial(
        pl.kernel,
        mesh=mesh,
        compiler_params=pltpu.CompilerParams(use_tc_tiling_on_sc=False),
        out_type=jax.ShapeDtypeStruct((B, L, D), jnp.float32),
        scratch_types=[
            pltpu.VMEM((CB, LP), jnp.int32),
            pltpu.VMEM((CB, LP), jnp.int32),
            pltpu.VMEM((CB, LP), jnp.int32),
            pltpu.VMEM((CB, LP), jnp.int32),
            pltpu.VMEM((LP,), jnp.int32),
            pltpu.VMEM((CB, LP, D), jnp.float32),
            pltpu.VMEM((CB, LP, D), jnp.float32),
            pltpu.VMEM_SHARED((L * D, D), jnp.float32),
            pltpu.SemaphoreType.DMA,
            pltpu.SemaphoreType.DMA,
            pltpu.SemaphoreType.DMA,
            pltpu.SemaphoreType.DMA,
            pltpu.SemaphoreType.DMA,
            pltpu.SemaphoreType.DMA,
        ],
    )
    def k(x_hbm, off_hbm, tab_hbm, out_hbm,
          idx0, idx1, idx2, idx3, off_v, rows0, rows1, tab_sh,
          sx0, sx1, sg0, sg1, so0, so1):
        idx = (idx0, idx1, idx2, idx3)
        rows = (rows0, rows1)
        sx = (sx0, sx1)
        sg = (sg0, sg1)
        so = (so0, so1)
        wid = lax.axis_index("s") * NC + lax.axis_index("c")
        brow = wid * BW
        pltpu.sync_copy(off_hbm, off_v)
        @pl.when(lax.axis_index("s") == 0)
        def _():
            pltpu.sync_copy(tab_hbm, tab_sh)

        plsc.subcore_barrier()

        def x_slice(c):
            return x_hbm.at[pl.ds(brow + c * CB, CB), :]

        def rows_src(s):
            return rows[s].at[:, pl.ds(0, L), :]

        def out_slice(c):
            return out_hbm.at[pl.ds(brow + c * CB, CB)]

        def fire_gathers(b, s):
            for r in range(CB):
                pltpu.async_copy(
                    tab_sh.at[idx[b].at[r]], rows[s].at[r], sg[s])

        def wait_gathers(b, s):
            for r in range(CB):
                pltpu.make_async_copy(
                    tab_sh.at[idx[b].at[r]], rows[s].at[r], sg[s]).wait()

        pltpu.async_copy(x_slice(0), idx[0], sx[0])

        @pl.loop(0, NCH // 4)
        def _(c4):
            for u in range(4):
                c = c4 * 4 + u
                i_cur = u & 3
                i_nxt = (u + 1) & 3
                i_prv = (u - 1) & 3
                r_cur = u & 1
                r_prv = (u - 1) & 1

                @pl.when(c + 1 < NCH)
                def _():
                    pltpu.async_copy(x_slice(c + 1), idx[i_nxt], sx[(u + 1) & 1])

                pltpu.make_async_copy(x_slice(c), idx[i_cur], sx[u & 1]).wait()

                @pl.loop(0, CB)
                def _(r):
                    for j in range(NVR):
                        s = pl.ds(j * LANES, LANES)
                        idx[i_cur][r, s] = idx[i_cur][r, s] + off_v[s]

                @pl.when(c >= 2)
                def _():
                    pltpu.make_async_copy(
                        rows_src(r_cur), out_slice(c), so[r_cur]).wait()

                fire_gathers(i_cur, r_cur)

                @pl.when(c > 0)
                def _():
                    wait_gathers(i_prv, r_prv)
                    pltpu.async_copy(rows_src(r_prv), out_slice(c - 1), so[r_prv])

        lastr = (NCH - 1) & 1
        wait_gathers((NCH - 1) & 3, lastr)
        pltpu.async_copy(rows_src(lastr), out_slice(NCH - 1), so[lastr])
        for b in (0, 1):
            pltpu.make_async_copy(rows_src(b), out_slice(0), so[b]).wait()

    out = k(x_pad, off, flat_tab)
    return out.reshape(B, L * D)


# speedup vs baseline: 3.7872x; 3.7872x over previous
import functools

import jax
import jax.numpy as jnp
from jax import lax
from jax.experimental import pallas as pl
from jax.experimental.pallas import tpu as pltpu
from jax.experimental.pallas import tpu_sc as plsc

LANES = 16


def kernel(x, tables):
    B, L = x.shape
    D = tables.shape[2]
    info = plsc.get_sparse_core_info()
    NC, NS = info.num_cores, info.num_subcores
    NW = NC * NS
    BW = B // NW
    CB = 32
    NCH = BW // CB
    CHUNK = CB * L
    NV = CHUNK // LANES

    flat_tab = tables.reshape(L * D, D)
    x_flat = x.reshape(B * L).astype(jnp.int32)
    off = jnp.tile(jnp.arange(L, dtype=jnp.int32) * D, CB)

    mesh = plsc.VectorSubcoreMesh(core_axis_name="c", subcore_axis_name="s")

    @functools.partial(
        pl.kernel,
        mesh=mesh,
        compiler_params=pltpu.CompilerParams(use_tc_tiling_on_sc=False),
        out_type=jax.ShapeDtypeStruct((B * L, D), jnp.float32),
        scratch_types=[
            pltpu.VMEM((CHUNK,), jnp.int32),
            pltpu.VMEM((CHUNK,), jnp.int32),
            pltpu.VMEM((CHUNK,), jnp.int32),
            pltpu.VMEM((CHUNK,), jnp.int32),
            pltpu.VMEM((CHUNK,), jnp.int32),
            pltpu.VMEM((CHUNK, D), jnp.float32),
            pltpu.VMEM((CHUNK, D), jnp.float32),
            pltpu.VMEM_SHARED((L * D, D), jnp.float32),
            pltpu.SemaphoreType.DMA,
            pltpu.SemaphoreType.DMA,
            pltpu.SemaphoreType.DMA,
            pltpu.SemaphoreType.DMA,
            pltpu.SemaphoreType.DMA,
            pltpu.SemaphoreType.DMA,
        ],
    )
    def k(x_hbm, off_hbm, tab_hbm, out_hbm,
          idx0, idx1, idx2, idx3, off_v, rows0, rows1, tab_sh,
          sx0, sx1, sg0, sg1, so0, so1):
        idx = (idx0, idx1, idx2, idx3)
        rows = (rows0, rows1)
        sx = (sx0, sx1)
        sg = (sg0, sg1)
        so = (so0, so1)
        wid = lax.axis_index("s") * NC + lax.axis_index("c")
        base = wid * (BW * L)
        pltpu.sync_copy(off_hbm, off_v)
        @pl.when(lax.axis_index("s") == 0)
        def _():
            pltpu.sync_copy(tab_hbm, tab_sh)

        plsc.subcore_barrier()
        pltpu.async_copy(x_hbm.at[pl.ds(base, CHUNK)], idx[0], sx[0])

        def x_slice(c):
            return x_hbm.at[pl.ds(base + c * CHUNK, CHUNK)]

        def out_slice(c):
            return out_hbm.at[pl.ds(base + c * CHUNK, CHUNK)]

        @pl.loop(0, NCH // 4)
        def _(c4):
            for u in range(4):
                c = c4 * 4 + u
                i_cur = u & 3
                i_nxt = (u + 1) & 3
                i_prv = (u - 1) & 3
                r_cur = u & 1
                r_prv = (u - 1) & 1

                @pl.when(c + 1 < NCH)
                def _():
                    pltpu.async_copy(x_slice(c + 1), idx[i_nxt], sx[(u + 1) & 1])

                pltpu.make_async_copy(x_slice(c), idx[i_cur], sx[u & 1]).wait()

                @pl.loop(0, NV, unroll=8)
                def _(j):
                    s = pl.ds(pl.multiple_of(j * LANES, LANES), LANES)
                    idx[i_cur][s] = idx[i_cur][s] + off_v[s]

                @pl.when(c >= 2)
                def _():
                    pltpu.make_async_copy(
                        rows[r_cur], out_slice(c), so[r_cur]).wait()

                pltpu.async_copy(tab_sh.at[idx[i_cur]], rows[r_cur], sg[r_cur])

                @pl.when(c > 0)
                def _():
                    pltpu.make_async_copy(
                        tab_sh.at[idx[i_prv]], rows[r_prv], sg[r_prv]).wait()
                    pltpu.async_copy(rows[r_prv], out_slice(c - 1), so[r_prv])

        lastr = (NCH - 1) & 1
        pltpu.make_async_copy(
            tab_sh.at[idx[(NCH - 1) & 3]], rows[lastr], sg[lastr]).wait()
        pltpu.async_copy(rows[lastr], out_slice(NCH - 1), so[lastr])
        for b in (0, 1):
            pltpu.make_async_copy(rows[b], out_slice(0), so[b]).wait()

    out = k(x_flat, off, flat_tab)
    return out.reshape(B, L * D)


# speedup vs baseline: 3.8184x; 1.0082x over previous
import functools

import jax
import jax.numpy as jnp
from jax import lax
from jax.experimental import pallas as pl
from jax.experimental.pallas import tpu as pltpu
from jax.experimental.pallas import tpu_sc as plsc

LANES = 16


def kernel(x, tables):
    B, L = x.shape
    D = tables.shape[2]
    info = plsc.get_sparse_core_info()
    NC, NS = info.num_cores, info.num_subcores
    NW = NC * NS
    BW = B // NW
    CB = 16
    NCH = BW // CB
    CHUNK = CB * L
    NV = CHUNK // LANES

    flat_tab = tables.reshape(L * D, D)
    x_flat = x.reshape(B * L).astype(jnp.int32)
    off = jnp.tile(jnp.arange(L, dtype=jnp.int32) * D, CB)

    mesh = plsc.VectorSubcoreMesh(core_axis_name="c", subcore_axis_name="s")

    @functools.partial(
        pl.kernel,
        mesh=mesh,
        compiler_params=pltpu.CompilerParams(use_tc_tiling_on_sc=False),
        out_type=jax.ShapeDtypeStruct((B * L, D), jnp.float32),
        scratch_types=[
            pltpu.VMEM((CHUNK,), jnp.int32),
            pltpu.VMEM((CHUNK,), jnp.int32),
            pltpu.VMEM((CHUNK,), jnp.int32),
            pltpu.VMEM((CHUNK,), jnp.int32),
            pltpu.VMEM((CHUNK,), jnp.int32),
            pltpu.VMEM((CHUNK, D), jnp.float32),
            pltpu.VMEM((CHUNK, D), jnp.float32),
            pltpu.VMEM_SHARED((L * D, D), jnp.float32),
            pltpu.SemaphoreType.DMA,
            pltpu.SemaphoreType.DMA,
            pltpu.SemaphoreType.DMA,
            pltpu.SemaphoreType.DMA,
            pltpu.SemaphoreType.DMA,
            pltpu.SemaphoreType.DMA,
        ],
    )
    def k(x_hbm, off_hbm, tab_hbm, out_hbm,
          idx0, idx1, idx2, idx3, off_v, rows0, rows1, tab_sh,
          sx0, sx1, sg0, sg1, so0, so1):
        idx = (idx0, idx1, idx2, idx3)
        rows = (rows0, rows1)
        sx = (sx0, sx1)
        sg = (sg0, sg1)
        so = (so0, so1)
        wid = lax.axis_index("s") * NC + lax.axis_index("c")
        base = wid * (BW * L)
        pltpu.sync_copy(off_hbm, off_v)
        @pl.when(lax.axis_index("s") == 0)
        def _():
            pltpu.sync_copy(tab_hbm, tab_sh)

        plsc.subcore_barrier()
        pltpu.async_copy(x_hbm.at[pl.ds(base, CHUNK)], idx[0], sx[0])

        def x_slice(c):
            return x_hbm.at[pl.ds(base + c * CHUNK, CHUNK)]

        def out_slice(c):
            return out_hbm.at[pl.ds(base + c * CHUNK, CHUNK)]

        @pl.loop(0, NCH // 4)
        def _(c4):
            for u in range(4):
                c = c4 * 4 + u
                i_cur = u & 3
                i_nxt = (u + 1) & 3
                i_prv = (u - 1) & 3
                r_cur = u & 1
                r_prv = (u - 1) & 1

                @pl.when(c + 1 < NCH)
                def _():
                    pltpu.async_copy(x_slice(c + 1), idx[i_nxt], sx[(u + 1) & 1])

                pltpu.make_async_copy(x_slice(c), idx[i_cur], sx[u & 1]).wait()

                @pl.loop(0, NV, unroll=8)
                def _(j):
                    s = pl.ds(pl.multiple_of(j * LANES, LANES), LANES)
                    idx[i_cur][s] = idx[i_cur][s] + off_v[s]

                @pl.when(c >= 2)
                def _():
                    pltpu.make_async_copy(
                        rows[r_cur], out_slice(c), so[r_cur]).wait()

                pltpu.async_copy(tab_sh.at[idx[i_cur]], rows[r_cur], sg[r_cur])

                @pl.when(c > 0)
                def _():
                    pltpu.make_async_copy(
                        tab_sh.at[idx[i_prv]], rows[r_prv], sg[r_prv]).wait()
                    pltpu.async_copy(rows[r_prv], out_slice(c - 1), so[r_prv])

        lastr = (NCH - 1) & 1
        pltpu.make_async_copy(
            tab_sh.at[idx[(NCH - 1) & 3]], rows[lastr], sg[lastr]).wait()
        pltpu.async_copy(rows[lastr], out_slice(NCH - 1), so[lastr])
        for b in (0, 1):
            pltpu.make_async_copy(rows[b], out_slice(0), so[b]).wait()

    out = k(x_flat, off, flat_tab)
    return out.reshape(B, L * D)
